# baseline (speedup 1.0000x reference)
import functools

import jax
import jax.numpy as jnp
from jax import lax
from jax.experimental import pallas as pl
from jax.experimental.pallas import tpu as pltpu
from jax.experimental.pallas import tpu_sc as plsc

DIM = 33
D3 = DIM * DIM * DIM
CH = 36864
NSAMP = 8
NPIX = 512 * 512
NCORES = 2
NSUBC = 16
NW = NCORES * NSUBC
WORKERS_PER_SAMP = NW // NSAMP
PIX_PER_W = NPIX // WORKERS_PER_SAMP
CHUNK = 1024
NCHUNK = PIX_PER_W // CHUNK
VPC = CHUNK // 16
INV_BIN = (DIM - 1) / 1.000001
OFFS = (0, 1, DIM, DIM + 1, DIM * DIM, DIM * DIM + 1,
        DIM * DIM + DIM, DIM * DIM + DIM + 1)
MAXBASE = (DIM - 2) * (1 + DIM + DIM * DIM) + 1


def _fuse_body(w_ref, lut_ref, out_ref):
    out_ref[0] = jnp.dot(w_ref[...], lut_ref[0],
                         preferred_element_type=jnp.float32)


def _fuse_luts(weights, luts_pad):
    num = weights.shape[1]
    nb = 4608
    grid = (3, CH // nb)
    return pl.pallas_call(
        _fuse_body,
        grid=grid,
        in_specs=[
            pl.BlockSpec((NSAMP, num), lambda c, n: (0, 0)),
            pl.BlockSpec((1, num, nb), lambda c, n: (c, 0, n)),
        ],
        out_specs=pl.BlockSpec((1, NSAMP, nb), lambda c, n: (c, 0, n)),
        out_shape=jax.ShapeDtypeStruct((3, NSAMP, CH), jnp.float32),
    )(weights, luts_pad)


ROWS_PER_W = 512 // WORKERS_PER_SAMP
RCHUNK = 8
NRCHUNK = ROWS_PER_W // RCHUNK
VPR = 512 // 16


def _interp_body(d3lut, x, out, lut_v, rb, gb, bb):
    wid = lax.axis_index("s") * NCORES + lax.axis_index("c")
    samp = wid // WORKERS_PER_SAMP
    quarter = wid % WORKERS_PER_SAMP
    row0 = quarter * ROWS_PER_W

    for c in range(3):
        pltpu.sync_copy(d3lut.at[pl.ds((c * NSAMP) * CH + samp * CH, CH)],
                        lut_v.at[pl.ds(c * CH, CH)])

    def chunk_body(t, carry):
        r0 = row0 + t * RCHUNK
        pltpu.sync_copy(x.at[samp, 0, pl.ds(r0, RCHUNK), :], rb)
        pltpu.sync_copy(x.at[samp, 1, pl.ds(r0, RCHUNK), :], gb)
        pltpu.sync_copy(x.at[samp, 2, pl.ds(r0, RCHUNK), :], bb)

        def vec_body(i, carry2):
            rr = i >> 5
            sl = pl.ds((i & 31) * 16, 16)
            rs = rb[rr, sl] * INV_BIN
            gs = gb[rr, sl] * INV_BIN
            bs = bb[rr, sl] * INV_BIN
            ri = jnp.minimum(rs.astype(jnp.int32), DIM - 2)
            gi = jnp.minimum(gs.astype(jnp.int32), DIM - 2)
            bi = jnp.minimum(bs.astype(jnp.int32), DIM - 2)
            rd = rs - ri.astype(jnp.float32)
            gd = gs - gi.astype(jnp.float32)
            bd = bs - bi.astype(jnp.float32)
            rd1 = 1.0 - rd
            gd1 = 1.0 - gd
            bd1 = 1.0 - bd
            a00 = rd1 * gd1
            a10 = rd * gd1
            a01 = rd1 * gd
            a11 = rd * gd
            w = (a00 * bd1, a10 * bd1, a01 * bd1, a11 * bd1,
                 a00 * bd, a10 * bd, a01 * bd, a11 * bd)
            base = ri + gi * DIM + bi * (DIM * DIM)
            idxs = (base, base + 1, base + 2, base + 3)
            accs = []
            for c in range(3):
                t = [w[k] * plsc.load_gather(
                        lut_v.at[pl.ds(c * CH + OFFS[k] - OFFS[k] % 8,
                                       MAXBASE + 4)],
                        [idxs[OFFS[k] % 8]])
                     for k in range(8)]
                accs.append(((t[0] + t[1]) + (t[2] + t[3]))
                            + ((t[4] + t[5]) + (t[6] + t[7])))
            rb[rr, sl] = accs[0]
            gb[rr, sl] = accs[1]
            bb[rr, sl] = accs[2]
            return carry2

        lax.fori_loop(0, RCHUNK * VPR, vec_body, 0, unroll=False)

        pltpu.sync_copy(rb, out.at[samp, 0, pl.ds(r0, RCHUNK), :])
        pltpu.sync_copy(gb, out.at[samp, 1, pl.ds(r0, RCHUNK), :])
        pltpu.sync_copy(bb, out.at[samp, 2, pl.ds(r0, RCHUNK), :])
        return carry

    lax.fori_loop(0, NRCHUNK, chunk_body, 0, unroll=False)


_sc_interp = functools.partial(
    pl.kernel,
    out_type=jax.ShapeDtypeStruct((NSAMP, 3, 512, 512), jnp.float32),
    mesh=plsc.VectorSubcoreMesh(core_axis_name="c", subcore_axis_name="s"),
    compiler_params=pltpu.CompilerParams(needs_layout_passes=False,
                                         use_tc_tiling_on_sc=True),
    scratch_types=[
        pltpu.VMEM((3 * CH,), jnp.float32),
        pltpu.VMEM((RCHUNK, 512), jnp.float32),
        pltpu.VMEM((RCHUNK, 512), jnp.float32),
        pltpu.VMEM((RCHUNK, 512), jnp.float32),
    ],
)(_interp_body)


def kernel(weights, x, LUTs):
    num = LUTs.shape[0]
    luts_flat = jnp.transpose(LUTs.reshape(num, 3, D3), (1, 0, 2))
    luts_pad = jnp.pad(luts_flat, ((0, 0), (0, 0), (0, CH - D3)))
    d3lut = _fuse_luts(weights, luts_pad)
    return _sc_interp(d3lut.reshape(-1), x)


# speedup vs baseline: 1.0703x; 1.0703x over previous
import functools

import jax
import jax.numpy as jnp
from jax import lax
from jax.experimental import pallas as pl
from jax.experimental.pallas import tpu as pltpu
from jax.experimental.pallas import tpu_sc as plsc

DIM = 33
D3 = DIM * DIM * DIM
CH = 36864
NSAMP = 8
NPIX = 512 * 512
NCORES = 2
NSUBC = 16
NW = NCORES * NSUBC
WORKERS_PER_SAMP = NW // NSAMP
PIX_PER_W = NPIX // WORKERS_PER_SAMP
CHUNK = 1024
NCHUNK = PIX_PER_W // CHUNK
VPC = CHUNK // 16
INV_BIN = (DIM - 1) / 1.000001
OFFS = (0, 1, DIM, DIM + 1, DIM * DIM, DIM * DIM + 1,
        DIM * DIM + DIM, DIM * DIM + DIM + 1)
MAXBASE = (DIM - 2) * (1 + DIM + DIM * DIM) + 1


def _fuse_body(w_ref, lut_ref, out_ref):
    out_ref[0] = jnp.dot(w_ref[...], lut_ref[0],
                         preferred_element_type=jnp.float32)


def _fuse_luts(weights, luts_pad):
    num = weights.shape[1]
    nb = 4608
    grid = (3, CH // nb)
    return pl.pallas_call(
        _fuse_body,
        grid=grid,
        in_specs=[
            pl.BlockSpec((NSAMP, num), lambda c, n: (0, 0)),
            pl.BlockSpec((1, num, nb), lambda c, n: (c, 0, n)),
        ],
        out_specs=pl.BlockSpec((1, NSAMP, nb), lambda c, n: (c, 0, n)),
        out_shape=jax.ShapeDtypeStruct((3, NSAMP, CH), jnp.float32),
    )(weights, luts_pad)


ROWS_PER_W = 512 // WORKERS_PER_SAMP
RCHUNK = 8
NRCHUNK = ROWS_PER_W // RCHUNK
VPR = 512 // 16


def _interp_body(d3lut, x, out, lut_v, rb, gb, bb):
    wid = lax.axis_index("s") * NCORES + lax.axis_index("c")
    samp = wid // WORKERS_PER_SAMP
    quarter = wid % WORKERS_PER_SAMP
    row0 = quarter * ROWS_PER_W

    for c in range(3):
        pltpu.sync_copy(d3lut.at[pl.ds((c * NSAMP) * CH + samp * CH, CH)],
                        lut_v.at[pl.ds(c * CH, CH)])

    def chunk_body(t, carry):
        r0 = row0 + t * RCHUNK
        pltpu.sync_copy(x.at[samp, 0, pl.ds(r0, RCHUNK), :], rb)
        pltpu.sync_copy(x.at[samp, 1, pl.ds(r0, RCHUNK), :], gb)
        pltpu.sync_copy(x.at[samp, 2, pl.ds(r0, RCHUNK), :], bb)

        def vec_body(i, carry2):
            rr = i >> 5
            sl = pl.ds((i & 31) * 16, 16)
            rs = rb[rr, sl] * INV_BIN
            gs = gb[rr, sl] * INV_BIN
            bs = bb[rr, sl] * INV_BIN
            ri = jnp.minimum(rs.astype(jnp.int32), DIM - 2)
            gi = jnp.minimum(gs.astype(jnp.int32), DIM - 2)
            bi = jnp.minimum(bs.astype(jnp.int32), DIM - 2)
            rd = rs - ri.astype(jnp.float32)
            gd = gs - gi.astype(jnp.float32)
            bd = bs - bi.astype(jnp.float32)
            rd1 = 1.0 - rd
            gd1 = 1.0 - gd
            bd1 = 1.0 - bd
            a00 = rd1 * gd1
            a10 = rd * gd1
            a01 = rd1 * gd
            a11 = rd * gd
            w = (a00 * bd1, a10 * bd1, a01 * bd1, a11 * bd1,
                 a00 * bd, a10 * bd, a01 * bd, a11 * bd)
            base = ri + gi * DIM + bi * (DIM * DIM)
            accs = []
            for c in range(3):
                bc = base + c * CH
                t = [w[k] * plsc.load_gather(lut_v, [bc + OFFS[k]])
                     for k in range(8)]
                accs.append(((t[0] + t[1]) + (t[2] + t[3]))
                            + ((t[4] + t[5]) + (t[6] + t[7])))
            rb[rr, sl] = accs[0]
            gb[rr, sl] = accs[1]
            bb[rr, sl] = accs[2]
            return carry2

        lax.fori_loop(0, RCHUNK * VPR, vec_body, 0, unroll=2)

        pltpu.sync_copy(rb, out.at[samp, 0, pl.ds(r0, RCHUNK), :])
        pltpu.sync_copy(gb, out.at[samp, 1, pl.ds(r0, RCHUNK), :])
        pltpu.sync_copy(bb, out.at[samp, 2, pl.ds(r0, RCHUNK), :])
        return carry

    lax.fori_loop(0, NRCHUNK, chunk_body, 0, unroll=False)


_sc_interp = functools.partial(
    pl.kernel,
    out_type=jax.ShapeDtypeStruct((NSAMP, 3, 512, 512), jnp.float32),
    mesh=plsc.VectorSubcoreMesh(core_axis_name="c", subcore_axis_name="s"),
    compiler_params=pltpu.CompilerParams(needs_layout_passes=False,
                                         use_tc_tiling_on_sc=True),
    scratch_types=[
        pltpu.VMEM((3 * CH,), jnp.float32),
        pltpu.VMEM((RCHUNK, 512), jnp.float32),
        pltpu.VMEM((RCHUNK, 512), jnp.float32),
        pltpu.VMEM((RCHUNK, 512), jnp.float32),
    ],
)(_interp_body)


def kernel(weights, x, LUTs):
    num = LUTs.shape[0]
    luts_flat = jnp.transpose(LUTs.reshape(num, 3, D3), (1, 0, 2))
    luts_pad = jnp.pad(luts_flat, ((0, 0), (0, 0), (0, CH - D3)))
    d3lut = _fuse_luts(weights, luts_pad)
    return _sc_interp(d3lut.reshape(-1), x)
